# baseline (speedup 1.0000x reference)
import jax
import jax.numpy as jnp
from jax.experimental import pallas as pl
from jax.experimental.pallas import tpu as pltpu

_BLK = 2048


def _bcast_copy_kernel(w_ref, o_ref):
    o_ref[...] = w_ref[...][None]


def kernel(x, weight):
    batch, seq = x.shape
    dim = weight.shape[1]
    return pl.pallas_call(
        _bcast_copy_kernel,
        grid=(seq // _BLK, batch),
        in_specs=[pl.BlockSpec((_BLK, dim), lambda j, b: (j, 0))],
        out_specs=pl.BlockSpec((1, _BLK, dim), lambda j, b: (b, j, 0)),
        out_shape=jax.ShapeDtypeStruct((batch, seq, dim), weight.dtype),
        compiler_params=pltpu.CompilerParams(
            dimension_semantics=("parallel", "arbitrary"),
        ),
    )(weight)


# speedup vs baseline: 1.0585x; 1.0585x over previous
import jax
import jax.numpy as jnp
from jax.experimental import pallas as pl
from jax.experimental.pallas import tpu as pltpu

_BLK = 1024


def _bcast_dma_kernel(w_ref, o_hbm, sems):
    j = pl.program_id(0)
    batch = o_hbm.shape[0]
    copies = [
        pltpu.make_async_copy(
            w_ref,
            o_hbm.at[b, pl.ds(j * _BLK, _BLK), :],
            sems.at[b],
        )
        for b in range(batch)
    ]
    for c in copies:
        c.start()
    for c in copies:
        c.wait()


def kernel(x, weight):
    batch, seq = x.shape
    dim = weight.shape[1]
    return pl.pallas_call(
        _bcast_dma_kernel,
        grid=(seq // _BLK,),
        in_specs=[pl.BlockSpec((_BLK, dim), lambda j: (j, 0))],
        out_specs=pl.BlockSpec(memory_space=pltpu.MemorySpace.HBM),
        out_shape=jax.ShapeDtypeStruct((batch, seq, dim), weight.dtype),
        scratch_shapes=[pltpu.SemaphoreType.DMA((batch,))],
        compiler_params=pltpu.CompilerParams(
            dimension_semantics=("arbitrary",),
        ),
    )(weight)
